# Initial kernel scaffold; baseline (speedup 1.0000x reference)
import jax
import jax.numpy as jnp
from jax.experimental import pallas as pl


def kernel(x, edge_index, edge_weights, We1, be1, Wn1, bn1, We2, be2, Wn2, bn2):
    raise NotImplementedError("write your pallas kernel here")


# speedup vs baseline: 2.9204x; 2.9204x over previous
import functools

import jax
import jax.numpy as jnp
from jax import lax
from jax.experimental import pallas as pl
from jax.experimental.pallas import tpu as pltpu
from jax.experimental.pallas import tpu_sc as plsc

N = 10000
E = 320000
D = 128
H = 128

NC = 2
NS = 16
NW = NC * NS
EPW = E // NW
CHUNK = 80
NCHUNK = EPW // CHUNK
NPAD = 10240
RPT = NPAD // NS
RBLK = 128


def _edge_body(xs_hbm, xdb_hbm, src_hbm, dst_hbm, ew_hbm, wrow_hbm, out_hbm,
               src_v, dst_v, ew_v, xsr, xdr, wrow_v, stage_v, agg_sh,
               sem_a, sem_b):
    c = lax.axis_index("c")
    s = lax.axis_index("s")
    wid = c * NS + s

    pltpu.sync_copy(wrow_hbm, wrow_v)
    wr = [wrow_v[pl.ds(16 * j, 16)] for j in range(8)]

    zero16 = jnp.zeros((16,), jnp.float32)

    def zrow(r, carry):
        for j in range(8):
            stage_v[r, pl.ds(16 * j, 16)] = zero16
        return carry

    lax.fori_loop(0, RBLK, zrow, 0)
    for i in range(RPT // RBLK):
        pltpu.sync_copy(stage_v, agg_sh.at[pl.ds(s * RPT + i * RBLK, RBLK)])
    plsc.subcore_barrier()

    def chunk(i, carry):
        base = pl.multiple_of(wid * EPW + i * CHUNK, 8)
        pltpu.sync_copy(src_hbm.at[pl.ds(base, CHUNK)], src_v)
        pltpu.sync_copy(dst_hbm.at[pl.ds(base, CHUNK)], dst_v)
        pltpu.sync_copy(ew_hbm.at[pl.ds(base, CHUNK)], ew_v)
        ga = pltpu.async_copy(xs_hbm.at[src_v], xsr, sem_a)
        gb = pltpu.async_copy(xdb_hbm.at[dst_v], xdr, sem_b)
        ga.wait()
        gb.wait()

        def group(g, gcarry):
            ewg = ew_v[pl.ds(16 * g, 16)]
            for k in range(16):
                e = 16 * g + k
                ewv = jnp.full((16,), ewg[k], jnp.float32)
                for j in range(8):
                    sl = pl.ds(16 * j, 16)
                    m = xsr[e, sl] + xdr[e, sl] + ewv * wr[j]
                    xsr[e, sl] = jnp.maximum(m, 0.0)
            return gcarry

        lax.fori_loop(0, CHUNK // 16, group, 0)
        pltpu.sync_copy(xsr, agg_sh.at[dst_v], add=True)
        return carry

    lax.fori_loop(0, NCHUNK, chunk, 0)
    plsc.subcore_barrier()

    for i in range(RPT // RBLK):
        r0 = s * RPT + i * RBLK
        pltpu.sync_copy(agg_sh.at[pl.ds(r0, RBLK)], stage_v)
        pltpu.sync_copy(stage_v, out_hbm.at[c, pl.ds(r0, RBLK)])


_edge_call = functools.partial(
    pl.kernel,
    mesh=plsc.VectorSubcoreMesh(core_axis_name="c", subcore_axis_name="s"),
    out_type=jax.ShapeDtypeStruct((NC, NPAD, H), jnp.float32),
    scratch_types=[
        pltpu.VMEM((CHUNK,), jnp.int32),
        pltpu.VMEM((CHUNK,), jnp.int32),
        pltpu.VMEM((CHUNK,), jnp.float32),
        pltpu.VMEM((CHUNK, H), jnp.float32),
        pltpu.VMEM((CHUNK, H), jnp.float32),
        pltpu.VMEM((H,), jnp.float32),
        pltpu.VMEM((RBLK, H), jnp.float32),
        pltpu.VMEM_SHARED((NPAD, H), jnp.float32),
        pltpu.SemaphoreType.DMA,
        pltpu.SemaphoreType.DMA,
    ],
)(_edge_body)


BN = 1000


def _pre_body(x_ref, ws_ref, wd_ref, be_ref, xs_ref, xdb_ref):
    xb = x_ref[...]
    xs_ref[...] = jnp.dot(xb, ws_ref[...], preferred_element_type=jnp.float32)
    xdb_ref[...] = (jnp.dot(xb, wd_ref[...], preferred_element_type=jnp.float32)
                    + be_ref[...])


def _mid_body(x_ref, agg_ref, wna_ref, wnb_ref, bn_ref, ws_ref, wd_ref, be_ref,
              h_ref, xs_ref, xdb_ref):
    a = agg_ref[0] + agg_ref[1]
    h = jnp.dot(x_ref[...], wna_ref[...], preferred_element_type=jnp.float32)
    h += jnp.dot(a, wnb_ref[...], preferred_element_type=jnp.float32)
    h = jnp.maximum(h + bn_ref[...], 0.0)
    h_ref[...] = h
    xs_ref[...] = jnp.dot(h, ws_ref[...], preferred_element_type=jnp.float32)
    xdb_ref[...] = (jnp.dot(h, wd_ref[...], preferred_element_type=jnp.float32)
                    + be_ref[...])


def _fin_body(x_ref, agg_ref, wna_ref, wnb_ref, bn_ref, out_ref):
    a = agg_ref[0] + agg_ref[1]
    h = jnp.dot(x_ref[...], wna_ref[...], preferred_element_type=jnp.float32)
    h += jnp.dot(a, wnb_ref[...], preferred_element_type=jnp.float32)
    out_ref[...] = jnp.maximum(h + bn_ref[...], 0.0)


_row_spec = pl.BlockSpec((BN, H), lambda i: (i, 0))
_w_spec = pl.BlockSpec((H, H), lambda i: (0, 0))
_b_spec = pl.BlockSpec((1, H), lambda i: (0, 0))
_agg_spec = pl.BlockSpec((NC, BN, H), lambda i: (0, i, 0))
_nh = jax.ShapeDtypeStruct((N, H), jnp.float32)

_pre_call = pl.pallas_call(
    _pre_body,
    grid=(N // BN,),
    in_specs=[_row_spec, _w_spec, _w_spec, _b_spec],
    out_specs=[_row_spec, _row_spec],
    out_shape=[_nh, _nh],
)

_mid_call = pl.pallas_call(
    _mid_body,
    grid=(N // BN,),
    in_specs=[_row_spec, _agg_spec, _w_spec, _w_spec, _b_spec, _w_spec, _w_spec,
              _b_spec],
    out_specs=[_row_spec, _row_spec, _row_spec],
    out_shape=[_nh, _nh, _nh],
)

_fin_call = pl.pallas_call(
    _fin_body,
    grid=(N // BN,),
    in_specs=[_row_spec, _agg_spec, _w_spec, _w_spec, _b_spec],
    out_specs=_row_spec,
    out_shape=_nh,
)


def kernel(x, edge_index, edge_weights, We1, be1, Wn1, bn1, We2, be2, Wn2, bn2):
    ei = edge_index.astype(jnp.int32)
    src, dst = ei[0], ei[1]
    ew = edge_weights.astype(jnp.float32)

    xs1, xdb1 = _pre_call(x, We1[:D], We1[D:2 * D], be1[None])
    agg1 = _edge_call(xs1, xdb1, src, dst, ew, We1[2 * D])
    x2, xs2, xdb2 = _mid_call(x, agg1, Wn1[:D], Wn1[D:], bn1[None],
                              We2[:H], We2[H:2 * H], be2[None])
    agg2 = _edge_call(xs2, xdb2, src, dst, ew, We2[2 * H])
    return _fin_call(x2, agg2, Wn2[:H], Wn2[H:], bn2[None])


# speedup vs baseline: 3.0934x; 1.0592x over previous
import functools

import jax
import jax.numpy as jnp
from jax import lax
from jax.experimental import pallas as pl
from jax.experimental.pallas import tpu as pltpu
from jax.experimental.pallas import tpu_sc as plsc

N = 10000
E = 320000
D = 128
H = 128

NC = 2
NS = 16
NW = NC * NS
EPW = E // NW
CHUNK = 80
NCHUNK = EPW // CHUNK
NPAD = 10240
RPT = NPAD // NS
RBLK = 128


def _edge_body(xs_hbm, xdb_hbm, src_hbm, dst_hbm, ew_hbm, wrow_hbm, out_hbm,
               src0, dst0, ew0, src1, dst1, ew1, xsr0, xdr0, xsr1, xdr1,
               wrow_v, agg_sh, isem0, isem1, gsem0, gsem1):
    c = lax.axis_index("c")
    s = lax.axis_index("s")
    wid = c * NS + s

    pltpu.sync_copy(wrow_hbm, wrow_v)

    idx =((src0, dst0, ew0, isem0), (src1, dst1, ew1, isem1))
    bufs = ((xsr0, xdr0, gsem0), (xsr1, xdr1, gsem1))

    zero16 = jnp.zeros((16,), jnp.float32)

    def zrow(r, carry):
        for j in range(8):
            xsr0[r, pl.ds(16 * j, 16)] = zero16
        return carry

    lax.fori_loop(0, CHUNK, zrow, 0)
    for i in range(RPT // CHUNK):
        pltpu.sync_copy(xsr0, agg_sh.at[pl.ds(s * RPT + i * CHUNK, CHUNK)])
    plsc.subcore_barrier()

    def fire_idx(i, p):
        src_b, dst_b, ew_b, sem = idx[p]
        base = pl.multiple_of(wid * EPW, 8) + pl.multiple_of(i * CHUNK, 8)
        pltpu.async_copy(src_hbm.at[pl.ds(base, CHUNK)], src_b, sem)
        pltpu.async_copy(dst_hbm.at[pl.ds(base, CHUNK)], dst_b, sem)
        pltpu.async_copy(ew_hbm.at[pl.ds(base, CHUNK)], ew_b, sem)

    def wait_idx(p):
        src_b, dst_b, ew_b, sem = idx[p]
        pltpu.make_async_copy(src_hbm.at[pl.ds(0, CHUNK)], src_b, sem).wait()
        pltpu.make_async_copy(dst_hbm.at[pl.ds(0, CHUNK)], dst_b, sem).wait()
        pltpu.make_async_copy(ew_hbm.at[pl.ds(0, CHUNK)], ew_b, sem).wait()

    def fire_gather(p):
        src_b, dst_b, _, _ = idx[p]
        xs_b, xd_b, sem = bufs[p]
        pltpu.async_copy(xs_hbm.at[src_b], xs_b, sem)
        pltpu.async_copy(xdb_hbm.at[dst_b], xd_b, sem)

    def process(p):
        src_b, dst_b, ew_b, _ = idx[p]
        xs_b, xd_b, sem = bufs[p]
        pltpu.make_async_copy(xs_hbm.at[src_b], xs_b, sem).wait()
        pltpu.make_async_copy(xdb_hbm.at[dst_b], xd_b, sem).wait()

        def group(g, gcarry):
            ewg = ew_b[pl.ds(16 * g, 16)]
            ewvs = [jnp.full((16,), ewg[k], jnp.float32) for k in range(16)]
            e0 = 16 * g

            def feat(jj, fcarry):
                sl = pl.ds(pl.multiple_of(16 * jj, 16), 16)
                wrj = wrow_v[sl]
                for k in range(16):
                    m = xs_b[e0 + k, sl] + xd_b[e0 + k, sl] + ewvs[k] * wrj
                    xs_b[e0 + k, sl] = jnp.maximum(m, 0.0)
                return fcarry

            lax.fori_loop(0, H // 16, feat, 0)
            return gcarry

        lax.fori_loop(0, CHUNK // 16, group, 0)
        pltpu.sync_copy(xs_b, agg_sh.at[dst_b], add=True)

    def body(i, p):
        wait_idx(1 - p)
        fire_gather(1 - p)
        process(p)
        fire_idx(jnp.minimum(i + 2, NCHUNK - 1), p)

    fire_idx(0, 0)
    fire_idx(1, 1)
    wait_idx(0)
    fire_gather(0)

    def pair(t, carry):
        body(2 * t, 0)
        body(2 * t + 1, 1)
        return carry

    lax.fori_loop(0, (NCHUNK - 1) // 2, pair, 0)
    body(NCHUNK - 1, 0)
    wait_idx(0)
    pltpu.make_async_copy(xs_hbm.at[src1], xsr1, gsem1).wait()
    pltpu.make_async_copy(xdb_hbm.at[dst1], xdr1, gsem1).wait()
    plsc.subcore_barrier()

    for i in range(RPT // CHUNK):
        r0 = s * RPT + i * CHUNK
        pltpu.sync_copy(agg_sh.at[pl.ds(r0, CHUNK)], xsr0)
        pltpu.sync_copy(xsr0, out_hbm.at[c, pl.ds(r0, CHUNK)])


_edge_call = functools.partial(
    pl.kernel,
    mesh=plsc.VectorSubcoreMesh(core_axis_name="c", subcore_axis_name="s"),
    out_type=jax.ShapeDtypeStruct((NC, NPAD, H), jnp.float32),
    scratch_types=[
        pltpu.VMEM((CHUNK,), jnp.int32),
        pltpu.VMEM((CHUNK,), jnp.int32),
        pltpu.VMEM((CHUNK,), jnp.float32),
        pltpu.VMEM((CHUNK,), jnp.int32),
        pltpu.VMEM((CHUNK,), jnp.int32),
        pltpu.VMEM((CHUNK,), jnp.float32),
        pltpu.VMEM((CHUNK, H), jnp.float32),
        pltpu.VMEM((CHUNK, H), jnp.float32),
        pltpu.VMEM((CHUNK, H), jnp.float32),
        pltpu.VMEM((CHUNK, H), jnp.float32),
        pltpu.VMEM((H,), jnp.float32),
        pltpu.VMEM_SHARED((NPAD, H), jnp.float32),
        pltpu.SemaphoreType.DMA,
        pltpu.SemaphoreType.DMA,
        pltpu.SemaphoreType.DMA,
        pltpu.SemaphoreType.DMA,
    ],
)(_edge_body)


BN = 1000


def _pre_body(x_ref, ws_ref, wd_ref, be_ref, xs_ref, xdb_ref):
    xb = x_ref[...]
    xs_ref[...] = jnp.dot(xb, ws_ref[...], preferred_element_type=jnp.float32)
    xdb_ref[...] = (jnp.dot(xb, wd_ref[...], preferred_element_type=jnp.float32)
                    + be_ref[...])


def _mid_body(x_ref, agg_ref, wna_ref, wnb_ref, bn_ref, ws_ref, wd_ref, be_ref,
              h_ref, xs_ref, xdb_ref):
    a = agg_ref[0] + agg_ref[1]
    h = jnp.dot(x_ref[...], wna_ref[...], preferred_element_type=jnp.float32)
    h += jnp.dot(a, wnb_ref[...], preferred_element_type=jnp.float32)
    h = jnp.maximum(h + bn_ref[...], 0.0)
    h_ref[...] = h
    xs_ref[...] = jnp.dot(h, ws_ref[...], preferred_element_type=jnp.float32)
    xdb_ref[...] = (jnp.dot(h, wd_ref[...], preferred_element_type=jnp.float32)
                    + be_ref[...])


def _fin_body(x_ref, agg_ref, wna_ref, wnb_ref, bn_ref, out_ref):
    a = agg_ref[0] + agg_ref[1]
    h = jnp.dot(x_ref[...], wna_ref[...], preferred_element_type=jnp.float32)
    h += jnp.dot(a, wnb_ref[...], preferred_element_type=jnp.float32)
    out_ref[...] = jnp.maximum(h + bn_ref[...], 0.0)


_row_spec = pl.BlockSpec((BN, H), lambda i: (i, 0))
_w_spec = pl.BlockSpec((H, H), lambda i: (0, 0))
_b_spec = pl.BlockSpec((1, H), lambda i: (0, 0))
_agg_spec = pl.BlockSpec((NC, BN, H), lambda i: (0, i, 0))
_nh = jax.ShapeDtypeStruct((N, H), jnp.float32)

_pre_call = pl.pallas_call(
    _pre_body,
    grid=(N // BN,),
    in_specs=[_row_spec, _w_spec, _w_spec, _b_spec],
    out_specs=[_row_spec, _row_spec],
    out_shape=[_nh, _nh],
)

_mid_call = pl.pallas_call(
    _mid_body,
    grid=(N // BN,),
    in_specs=[_row_spec, _agg_spec, _w_spec, _w_spec, _b_spec, _w_spec, _w_spec,
              _b_spec],
    out_specs=[_row_spec, _row_spec, _row_spec],
    out_shape=[_nh, _nh, _nh],
)

_fin_call = pl.pallas_call(
    _fin_body,
    grid=(N // BN,),
    in_specs=[_row_spec, _agg_spec, _w_spec, _w_spec, _b_spec],
    out_specs=_row_spec,
    out_shape=_nh,
)


def kernel(x, edge_index, edge_weights, We1, be1, Wn1, bn1, We2, be2, Wn2, bn2):
    ei = edge_index.astype(jnp.int32)
    src, dst = ei[0], ei[1]
    ew = edge_weights.astype(jnp.float32)

    xs1, xdb1 = _pre_call(x, We1[:D], We1[D:2 * D], be1[None])
    agg1 = _edge_call(xs1, xdb1, src, dst, ew, We1[2 * D])
    x2, xs2, xdb2 = _mid_call(x, agg1, Wn1[:D], Wn1[D:], bn1[None],
                              We2[:H], We2[H:2 * H], be2[None])
    agg2 = _edge_call(xs2, xdb2, src, dst, ew, We2[2 * H])
    return _fin_call(x2, agg2, Wn2[:H], Wn2[H:], bn2[None])
